# baseline (speedup 1.0000x reference)
import functools

import jax
import jax.numpy as jnp
from jax import lax
from jax.experimental import pallas as pl
from jax.experimental.pallas import tpu as pltpu
from jax.experimental.pallas import tpu_sc as plsc

_N = 10000
_D = 256
_C = 4
_V = 4

_BN = 1000
_NP = _N // _BN
_G = 32
_NW = 32
_BPW = 40
_NBLKP = _NW * _BPW


def _dot_t(x, w):
    return lax.dot_general(x, w, (((1,), (1,)), ((), ())),
                           preferred_element_type=jnp.float32)


def _combine(x, w1, b1, w2, b2):
    y = jax.nn.sigmoid(_dot_t(x, w1) + b1) + (_dot_t(x, w2) + b2)
    nrm = jnp.sqrt(jnp.sum(y * y, axis=-1, keepdims=True))
    return y / (nrm + 1e-8)


def _table_body(v_ref, wn_ref, bn_ref, w1_ref, b1_ref, w2_ref, b2_ref,
                out_ref):
    v = v_ref[...]
    nv = _dot_t(v, wn_ref[...]) + bn_ref[...]
    w1, b1 = w1_ref[...], b1_ref[...]
    w2, b2 = w2_ref[...], b2_ref[...]
    out_ref[0] = _combine(v, w1, b1, w2, b2)
    out_ref[1] = _combine(nv, w1, b1, w2, b2)


def _build_table(variables, wn, bn, w1, b1, w2, b2):
    full = pl.BlockSpec((_D, _D), lambda i: (0, 0))
    row = pl.BlockSpec((1, _D), lambda i: (0, 0))
    return pl.pallas_call(
        _table_body,
        grid=(_NP,),
        in_specs=[pl.BlockSpec((_BN, _D), lambda i: (i, 0)),
                  full, row, full, row, full, row],
        out_specs=pl.BlockSpec((2, _BN, _D), lambda i: (0, i, 0)),
        out_shape=jax.ShapeDtypeStruct((2, _N, _D), jnp.float32),
    )(variables, wn, bn.reshape(1, _D), w1, b1.reshape(1, _D),
      w2, b2.reshape(1, _D))


def _accumulate(r, a):
    def clause(g, c2):
        for ch in range(_D // 16):
            s = pl.ds(ch * 16, 16)
            a[g, s] = (r[_V * g, s] + r[_V * g + 1, s]
                       + r[_V * g + 2, s] + r[_V * g + 3, s])
        return c2

    lax.fori_loop(0, _G, clause, 0)


def _sc_body(table, idx, out, idx_v, rows_v, acc_v, gs0, gs1, os0, os1):
    wid = lax.axis_index("s") * 2 + lax.axis_index("c")
    base = wid * _BPW

    pltpu.sync_copy(idx.at[wid], idx_v)
    pltpu.async_copy(table.at[idx_v.at[0]], rows_v.at[0], gs0)

    def pair(p, carry):
        b0 = 2 * p
        pltpu.make_async_copy(table.at[idx_v.at[b0]], rows_v.at[0], gs0).wait()
        pltpu.async_copy(table.at[idx_v.at[b0 + 1]], rows_v.at[1], gs1)

        @pl.when(p > 0)
        def _():
            pltpu.make_async_copy(
                acc_v.at[0], out.at[pl.ds((base + b0 - 2) * _G, _G)],
                os0).wait()

        _accumulate(rows_v.at[0], acc_v.at[0])
        pltpu.async_copy(acc_v.at[0], out.at[pl.ds((base + b0) * _G, _G)], os0)

        pltpu.make_async_copy(
            table.at[idx_v.at[b0 + 1]], rows_v.at[1], gs1).wait()

        @pl.when(b0 + 2 < _BPW)
        def _():
            pltpu.async_copy(table.at[idx_v.at[b0 + 2]], rows_v.at[0], gs0)

        @pl.when(p > 0)
        def _():
            pltpu.make_async_copy(
                acc_v.at[1], out.at[pl.ds((base + b0 - 1) * _G, _G)],
                os1).wait()

        _accumulate(rows_v.at[1], acc_v.at[1])
        pltpu.async_copy(
            acc_v.at[1], out.at[pl.ds((base + b0 + 1) * _G, _G)], os1)
        return carry

    lax.fori_loop(0, _BPW // 2, pair, 0)
    pltpu.make_async_copy(
        acc_v.at[0], out.at[pl.ds((base + _BPW - 2) * _G, _G)], os0).wait()
    pltpu.make_async_copy(
        acc_v.at[1], out.at[pl.ds((base + _BPW - 1) * _G, _G)], os1).wait()


@functools.cache
def _sc_gather_sum_fn():
    mesh = plsc.VectorSubcoreMesh(core_axis_name="c", subcore_axis_name="s")
    return pl.kernel(
        _sc_body,
        mesh=mesh,
        out_type=jax.ShapeDtypeStruct((_NBLKP * _G, _D), jnp.float32),
        scratch_types=[
            pltpu.VMEM((_BPW, _G * _V), jnp.int32),
            pltpu.VMEM((2, _G * _V, _D), jnp.float32),
            pltpu.VMEM((2, _G, _D), jnp.float32),
            pltpu.SemaphoreType.DMA,
            pltpu.SemaphoreType.DMA,
            pltpu.SemaphoreType.DMA,
            pltpu.SemaphoreType.DMA,
        ],
    )


def _update_body(ce0_ref, ce1_ref, ce2_ref, ce3_ref, v_ref,
                 w1_ref, b1_ref, w2_ref, b2_ref,
                 wz_ref, uz_ref, wr_ref, ur_ref, w_ref, u_ref, out_ref):
    w1, b1 = w1_ref[...], b1_ref[...]
    w2, b2 = w2_ref[...], b2_ref[...]
    av = _combine(ce0_ref[...], w1, b1, w2, b2)
    for ce_ref in (ce1_ref, ce2_ref, ce3_ref):
        av = av + _combine(ce_ref[...], w1, b1, w2, b2)
    x = v_ref[...]
    z = jax.nn.sigmoid(_dot_t(av, wz_ref[...]) + _dot_t(x, uz_ref[...]))
    r = jax.nn.sigmoid(_dot_t(av, wr_ref[...]) + _dot_t(x, ur_ref[...]))
    h_t = jnp.tanh(_dot_t(av, w_ref[...]) + _dot_t(r * x, u_ref[...]))
    out_ref[...] = (1.0 - z) * x + z * h_t


def _update(ce_pad, variables, w1, b1, w2, b2, wz, uz, wr, ur, w, u):
    full = pl.BlockSpec((_D, _D), lambda i: (0, 0))
    row = pl.BlockSpec((1, _D), lambda i: (0, 0))

    def plane_spec(c):
        return pl.BlockSpec((_BN, _D), lambda i, c=c: (c * _NP + i, 0))

    return pl.pallas_call(
        _update_body,
        grid=(_NP,),
        in_specs=[plane_spec(0), plane_spec(1), plane_spec(2), plane_spec(3),
                  pl.BlockSpec((_BN, _D), lambda i: (i, 0)),
                  full, row, full, row, full, full, full, full, full, full],
        out_specs=pl.BlockSpec((_BN, _D), lambda i: (i, 0)),
        out_shape=jax.ShapeDtypeStruct((_N, _D), jnp.float32),
    )(ce_pad, ce_pad, ce_pad, ce_pad, variables,
      w1, b1.reshape(1, _D), w2, b2.reshape(1, _D), wz, uz, wr, ur, w, u)


def kernel(variables, lits, Wn, bn, W1v, b1v, W2v, b2v, W1c, b1c, W2c, b2c,
           Wz, Uz, Wr, Ur, W, U):
    idx_flat = jnp.transpose(lits.astype(jnp.int32), (1, 0, 2)).reshape(-1)
    idx_blk = jnp.pad(idx_flat, (0, _NBLKP * _G * _V - idx_flat.shape[0])
                      ).reshape(_NW, _BPW, _G * _V)
    y_table = _build_table(variables, Wn, bn, W1v, b1v, W2v, b2v)
    ce_pad = _sc_gather_sum_fn()(y_table.reshape(2 * _N, _D), idx_blk)
    return _update(ce_pad, variables,
                   W1c, b1c, W2c, b2c, Wz, Uz, Wr, Ur, W, U)


# speedup vs baseline: 1.6576x; 1.6576x over previous
import functools

import jax
import jax.numpy as jnp
from jax import lax
from jax.experimental import pallas as pl
from jax.experimental.pallas import tpu as pltpu
from jax.experimental.pallas import tpu_sc as plsc

_N = 10000
_D = 256
_C = 4
_V = 4

_BN = 1000
_NP = _N // _BN
_G = 32
_NW = 32
_BPW = 40
_NBLKP = _NW * _BPW


def _dot_t(x, w):
    return lax.dot_general(x, w, (((1,), (1,)), ((), ())),
                           preferred_element_type=jnp.float32)


def _combine(x, w1, b1, w2, b2):
    y = jax.nn.sigmoid(_dot_t(x, w1) + b1) + (_dot_t(x, w2) + b2)
    nrm = jnp.sqrt(jnp.sum(y * y, axis=-1, keepdims=True))
    return y / (nrm + 1e-8)


def _table_body(v_ref, wn_ref, bn_ref, w1_ref, b1_ref, w2_ref, b2_ref,
                out_ref):
    v = v_ref[...]
    nv = _dot_t(v, wn_ref[...]) + bn_ref[...]
    w1, b1 = w1_ref[...], b1_ref[...]
    w2, b2 = w2_ref[...], b2_ref[...]
    out_ref[0] = _combine(v, w1, b1, w2, b2)
    out_ref[1] = _combine(nv, w1, b1, w2, b2)


def _build_table(variables, wn, bn, w1, b1, w2, b2):
    full = pl.BlockSpec((_D, _D), lambda i: (0, 0))
    row = pl.BlockSpec((1, _D), lambda i: (0, 0))
    return pl.pallas_call(
        _table_body,
        grid=(_NP,),
        in_specs=[pl.BlockSpec((_BN, _D), lambda i: (i, 0)),
                  full, row, full, row, full, row],
        out_specs=pl.BlockSpec((2, _BN, _D), lambda i: (0, i, 0)),
        out_shape=jax.ShapeDtypeStruct((2, _N, _D), jnp.float32),
    )(variables, wn, bn.reshape(1, _D), w1, b1.reshape(1, _D),
      w2, b2.reshape(1, _D))


def _accumulate(r, a):
    def clause(g, c2):
        for ch in range(_D // 16):
            s = pl.ds(ch * 16, 16)
            a[g, s] = (r[_V * g, s] + r[_V * g + 1, s]
                       + r[_V * g + 2, s] + r[_V * g + 3, s])
        return c2

    lax.fori_loop(0, _G, clause, 0)


def _sc_body(table, idx, out, idx_v, rows_v, acc_v, gs0, gs1, os0, os1):
    wid = lax.axis_index("s") * 2 + lax.axis_index("c")
    base = wid * _BPW

    pltpu.sync_copy(idx.at[wid], idx_v)
    pltpu.async_copy(table.at[idx_v.at[0]], rows_v.at[0], gs0)

    def pair(p, carry):
        b0 = 2 * p
        pltpu.make_async_copy(table.at[idx_v.at[b0]], rows_v.at[0], gs0).wait()
        pltpu.async_copy(table.at[idx_v.at[b0 + 1]], rows_v.at[1], gs1)

        @pl.when(p > 0)
        def _():
            pltpu.make_async_copy(
                acc_v.at[0], out.at[pl.ds((base + b0 - 2) * _G, _G)],
                os0).wait()

        _accumulate(rows_v.at[0], acc_v.at[0])
        pltpu.async_copy(acc_v.at[0], out.at[pl.ds((base + b0) * _G, _G)], os0)

        pltpu.make_async_copy(
            table.at[idx_v.at[b0 + 1]], rows_v.at[1], gs1).wait()

        @pl.when(b0 + 2 < _BPW)
        def _():
            pltpu.async_copy(table.at[idx_v.at[b0 + 2]], rows_v.at[0], gs0)

        @pl.when(p > 0)
        def _():
            pltpu.make_async_copy(
                acc_v.at[1], out.at[pl.ds((base + b0 - 1) * _G, _G)],
                os1).wait()

        _accumulate(rows_v.at[1], acc_v.at[1])
        pltpu.async_copy(
            acc_v.at[1], out.at[pl.ds((base + b0 + 1) * _G, _G)], os1)
        return carry

    lax.fori_loop(0, _BPW // 2, pair, 0)
    pltpu.make_async_copy(
        acc_v.at[0], out.at[pl.ds((base + _BPW - 2) * _G, _G)], os0).wait()
    pltpu.make_async_copy(
        acc_v.at[1], out.at[pl.ds((base + _BPW - 1) * _G, _G)], os1).wait()


@functools.cache
def _sc_gather_sum_fn():
    mesh = plsc.VectorSubcoreMesh(core_axis_name="c", subcore_axis_name="s")
    return pl.kernel(
        _sc_body,
        mesh=mesh,
        out_type=jax.ShapeDtypeStruct((_NBLKP * _G, _D), jnp.float32),
        scratch_types=[
            pltpu.VMEM((_BPW, _G * _V), jnp.int32),
            pltpu.VMEM((2, _G * _V, _D), jnp.float32),
            pltpu.VMEM((2, _G, _D), jnp.float32),
            pltpu.SemaphoreType.DMA,
            pltpu.SemaphoreType.DMA,
            pltpu.SemaphoreType.DMA,
            pltpu.SemaphoreType.DMA,
        ],
    )


def _update_body(ce0_ref, ce1_ref, ce2_ref, ce3_ref, v_ref,
                 w1_ref, b1_ref, w2_ref, b2_ref,
                 wz_ref, uz_ref, wr_ref, ur_ref, w_ref, u_ref, out_ref):
    w1, b1 = w1_ref[...], b1_ref[...]
    w2, b2 = w2_ref[...], b2_ref[...]
    av = _combine(ce0_ref[...], w1, b1, w2, b2)
    for ce_ref in (ce1_ref, ce2_ref, ce3_ref):
        av = av + _combine(ce_ref[...], w1, b1, w2, b2)
    x = v_ref[...]
    z = jax.nn.sigmoid(_dot_t(av, wz_ref[...]) + _dot_t(x, uz_ref[...]))
    r = jax.nn.sigmoid(_dot_t(av, wr_ref[...]) + _dot_t(x, ur_ref[...]))
    h_t = jnp.tanh(_dot_t(av, w_ref[...]) + _dot_t(r * x, u_ref[...]))
    out_ref[...] = (1.0 - z) * x + z * h_t


def _update(ce_pad, variables, w1, b1, w2, b2, wz, uz, wr, ur, w, u):
    full = pl.BlockSpec((_D, _D), lambda i: (0, 0))
    row = pl.BlockSpec((1, _D), lambda i: (0, 0))

    def plane_spec(c):
        return pl.BlockSpec((_BN, _D), lambda i, c=c: (c * _NP + i, 0))

    return pl.pallas_call(
        _update_body,
        grid=(_NP,),
        in_specs=[plane_spec(0), plane_spec(1), plane_spec(2), plane_spec(3),
                  pl.BlockSpec((_BN, _D), lambda i: (i, 0)),
                  full, row, full, row, full, full, full, full, full, full],
        out_specs=pl.BlockSpec((_BN, _D), lambda i: (i, 0)),
        out_shape=jax.ShapeDtypeStruct((_N, _D), jnp.float32),
    )(ce_pad, ce_pad, ce_pad, ce_pad, variables,
      w1, b1.reshape(1, _D), w2, b2.reshape(1, _D), wz, uz, wr, ur, w, u)


def kernel(variables, lits, Wn, bn, W1v, b1v, W2v, b2v, W1c, b1c, W2c, b2c,
           Wz, Uz, Wr, Ur, W, U):
    idx_flat = jnp.transpose(lits.astype(jnp.int32), (1, 0, 2)).reshape(-1)
    npad = _NBLKP * _G * _V - idx_flat.shape[0]
    idx_blk = jnp.concatenate(
        [idx_flat, jnp.arange(npad, dtype=jnp.int32)]
    ).reshape(_NW, _BPW, _G * _V)
    y_table = _build_table(variables, Wn, bn, W1v, b1v, W2v, b2v)
    ce_pad = _sc_gather_sum_fn()(y_table.reshape(2 * _N, _D), idx_blk)
    return _update(ce_pad, variables,
                   W1c, b1c, W2c, b2c, Wz, Uz, Wr, Ur, W, U)


# speedup vs baseline: 1.6657x; 1.0049x over previous
import functools

import jax
import jax.numpy as jnp
from jax import lax
from jax.experimental import pallas as pl
from jax.experimental.pallas import tpu as pltpu
from jax.experimental.pallas import tpu_sc as plsc

_N = 10000
_D = 256
_C = 4
_V = 4

_BN = 2000
_NP = _N // _BN
_G = 32
_NW = 32
_BPW = 40
_NBLKP = _NW * _BPW
_DW = _D // 2


def _dot_t(x, w):
    return lax.dot_general(x, w, (((1,), (1,)), ((), ())),
                           preferred_element_type=jnp.float32)


def _combine(x, w1, b1, w2, b2):
    y = jax.nn.sigmoid(_dot_t(x, w1) + b1) + (_dot_t(x, w2) + b2)
    nrm = jnp.sqrt(jnp.sum(y * y, axis=-1, keepdims=True))
    return y / (nrm + 1e-8)


def _table_body(v_ref, wn_ref, bn_ref, w1_ref, b1_ref, w2_ref, b2_ref,
                out_ref):
    v = v_ref[...]
    nv = _dot_t(v, wn_ref[...]) + bn_ref[...]
    w1, b1 = w1_ref[...], b1_ref[...]
    w2, b2 = w2_ref[...], b2_ref[...]
    out_ref[0] = _combine(v, w1, b1, w2, b2)
    out_ref[1] = _combine(nv, w1, b1, w2, b2)


def _build_table(variables, wn, bn, w1, b1, w2, b2):
    full = pl.BlockSpec((_D, _D), lambda i: (0, 0))
    row = pl.BlockSpec((1, _D), lambda i: (0, 0))
    return pl.pallas_call(
        _table_body,
        grid=(_NP,),
        in_specs=[pl.BlockSpec((_BN, _D), lambda i: (i, 0)),
                  full, row, full, row, full, row],
        out_specs=pl.BlockSpec((2, _BN, _D), lambda i: (0, i, 0)),
        out_shape=jax.ShapeDtypeStruct((2, _N, _D), jnp.float32),
    )(variables, wn, bn.reshape(1, _D), w1, b1.reshape(1, _D),
      w2, b2.reshape(1, _D))


def _accumulate(r, a):
    def clause(g, c2):
        for ch in range(_D // 16):
            s = pl.ds(ch * 16, 16)
            a[g, s] = (r[_V * g, s] + r[_V * g + 1, s]
                       + r[_V * g + 2, s] + r[_V * g + 3, s])
        return c2

    lax.fori_loop(0, _G, clause, 0)


def _sc_body(table, idx, out, idx_v, rows_v, acc_v, gs0, gs1, os0, os1):
    wid = lax.axis_index("s") * 2 + lax.axis_index("c")
    base = wid * _BPW

    pltpu.sync_copy(idx.at[wid], idx_v)
    pltpu.async_copy(table.at[idx_v.at[0]], rows_v.at[0], gs0)

    def pair(p, carry):
        b0 = 2 * p
        pltpu.async_copy(table.at[idx_v.at[b0 + 1]], rows_v.at[1], gs1)
        pltpu.make_async_copy(table.at[idx_v.at[b0]], rows_v.at[0], gs0).wait()

        @pl.when(p > 0)
        def _():
            pltpu.make_async_copy(
                acc_v.at[0], out.at[pl.ds((base + b0 - 2) * _G, _G)],
                os0).wait()

        _accumulate(rows_v.at[0], acc_v.at[0])
        pltpu.async_copy(acc_v.at[0], out.at[pl.ds((base + b0) * _G, _G)], os0)

        @pl.when(b0 + 2 < _BPW)
        def _():
            pltpu.async_copy(table.at[idx_v.at[b0 + 2]], rows_v.at[0], gs0)

        pltpu.make_async_copy(
            table.at[idx_v.at[b0 + 1]], rows_v.at[1], gs1).wait()

        @pl.when(p > 0)
        def _():
            pltpu.make_async_copy(
                acc_v.at[1], out.at[pl.ds((base + b0 - 1) * _G, _G)],
                os1).wait()

        _accumulate(rows_v.at[1], acc_v.at[1])
        pltpu.async_copy(
            acc_v.at[1], out.at[pl.ds((base + b0 + 1) * _G, _G)], os1)
        return carry

    lax.fori_loop(0, _BPW // 2, pair, 0)
    pltpu.make_async_copy(
        acc_v.at[0], out.at[pl.ds((base + _BPW - 2) * _G, _G)], os0).wait()
    pltpu.make_async_copy(
        acc_v.at[1], out.at[pl.ds((base + _BPW - 1) * _G, _G)], os1).wait()


@functools.cache
def _sc_gather_sum_fn():
    mesh = plsc.VectorSubcoreMesh(core_axis_name="c", subcore_axis_name="s")
    return pl.kernel(
        _sc_body,
        mesh=mesh,
        out_type=jax.ShapeDtypeStruct((_NBLKP * _G, _D), jnp.float32),
        scratch_types=[
            pltpu.VMEM((_BPW, _G * _V), jnp.int32),
            pltpu.VMEM((2, _G * _V, _D), jnp.float32),
            pltpu.VMEM((2, _G, _D), jnp.float32),
            pltpu.SemaphoreType.DMA,
            pltpu.SemaphoreType.DMA,
            pltpu.SemaphoreType.DMA,
            pltpu.SemaphoreType.DMA,
        ],
    )


def _update_body(ce0_ref, ce1_ref, ce2_ref, ce3_ref, v_ref,
                 w1_ref, b1_ref, w2_ref, b2_ref,
                 wz_ref, uz_ref, wr_ref, ur_ref, w_ref, u_ref, out_ref):
    w1, b1 = w1_ref[...], b1_ref[...]
    w2, b2 = w2_ref[...], b2_ref[...]
    av = _combine(ce0_ref[...], w1, b1, w2, b2)
    for ce_ref in (ce1_ref, ce2_ref, ce3_ref):
        av = av + _combine(ce_ref[...], w1, b1, w2, b2)
    x = v_ref[...]
    z = jax.nn.sigmoid(_dot_t(av, wz_ref[...]) + _dot_t(x, uz_ref[...]))
    r = jax.nn.sigmoid(_dot_t(av, wr_ref[...]) + _dot_t(x, ur_ref[...]))
    h_t = jnp.tanh(_dot_t(av, w_ref[...]) + _dot_t(r * x, u_ref[...]))
    out_ref[...] = (1.0 - z) * x + z * h_t


def _update(ce_pad, variables, w1, b1, w2, b2, wz, uz, wr, ur, w, u):
    full = pl.BlockSpec((_D, _D), lambda i: (0, 0))
    row = pl.BlockSpec((1, _D), lambda i: (0, 0))

    def plane_spec(c):
        return pl.BlockSpec((_BN, _D), lambda i, c=c: (c * _NP + i, 0))

    return pl.pallas_call(
        _update_body,
        grid=(_NP,),
        in_specs=[plane_spec(0), plane_spec(1), plane_spec(2), plane_spec(3),
                  pl.BlockSpec((_BN, _D), lambda i: (i, 0)),
                  full, row, full, row, full, full, full, full, full, full],
        out_specs=pl.BlockSpec((_BN, _D), lambda i: (i, 0)),
        out_shape=jax.ShapeDtypeStruct((_N, _D), jnp.float32),
    )(ce_pad, ce_pad, ce_pad, ce_pad, variables,
      w1, b1.reshape(1, _D), w2, b2.reshape(1, _D), wz, uz, wr, ur, w, u)


def kernel(variables, lits, Wn, bn, W1v, b1v, W2v, b2v, W1c, b1c, W2c, b2c,
           Wz, Uz, Wr, Ur, W, U):
    idx_flat = jnp.transpose(lits.astype(jnp.int32), (1, 0, 2)).reshape(-1)
    npad = _NBLKP * _G * _V - idx_flat.shape[0]
    idx_blk = jnp.concatenate(
        [idx_flat, jnp.arange(npad, dtype=jnp.int32)]
    ).reshape(_NW, _BPW, _G * _V)
    y_table = _build_table(variables, Wn, bn, W1v, b1v, W2v, b2v)
    ce_pad = _sc_gather_sum_fn()(y_table.reshape(2 * _N, _D), idx_blk)
    return _update(ce_pad, variables,
                   W1c, b1c, W2c, b2c, Wz, Uz, Wr, Ur, W, U)
